# baseline (device time: 50482 ns/iter reference)
import jax
import jax.numpy as jnp
from jax import lax
from jax.experimental import pallas as pl
from jax.experimental.pallas import tpu as pltpu

A_CHUNK_ROWS = (512, 512, 384, 128)
N_A_CHUNKS = len(A_CHUNK_ROWS)

_SEM_B = 0
_SEM_A0 = 1
_SEM_FWD = _SEM_A0 + N_A_CHUNKS
_N_SEMS = _SEM_FWD + 1


def kernel(A, B):
    m, k = A.shape
    k2, n = B.shape
    assert k == k2
    assert sum(A_CHUNK_ROWS) == m
    a_offs = [sum(A_CHUNK_ROWS[:c]) for c in range(N_A_CHUNKS)]
    nh = n // 2

    def body(a_ref, b_ref, out_ref,
             acc, a_bf16, b_bf16, a_nbr, b_nbr,
             send_sems, recv_sems, out_sems):
        my_x = lax.axis_index("x")
        my_y = lax.axis_index("y")
        ynbr = (my_x, 1 - my_y)
        xnbr = (1 - my_x, my_y)
        col0 = my_x * nh
        col1 = (1 - my_x) * nh

        barrier_sem = pltpu.get_barrier_semaphore()
        for nbr in (ynbr, xnbr):
            pl.semaphore_signal(
                barrier_sem, inc=1,
                device_id=nbr, device_id_type=pl.DeviceIdType.MESH,
            )

        b_bf16[:, pl.ds(col0, nh)] = (
            b_ref[:, pl.ds(col0, nh)].astype(jnp.bfloat16)
        )
        pl.semaphore_wait(barrier_sem, 2)

        rdma_b = pltpu.make_async_remote_copy(
            src_ref=b_bf16.at[:, pl.ds(col0, nh)],
            dst_ref=b_nbr.at[:, pl.ds(col0, nh)],
            send_sem=send_sems.at[_SEM_B],
            recv_sem=recv_sems.at[_SEM_B],
            device_id=ynbr,
            device_id_type=pl.DeviceIdType.MESH,
        )
        rdma_b.start()

        rdma_a = []
        for c in range(N_A_CHUNKS):
            o, mc = a_offs[c], A_CHUNK_ROWS[c]
            a_bf16[pl.ds(o, mc), :] = (
                a_ref[pl.ds(o, mc), :].astype(jnp.bfloat16)
            )
            rdma = pltpu.make_async_remote_copy(
                src_ref=a_bf16.at[pl.ds(o, mc), :],
                dst_ref=a_nbr.at[pl.ds(o, mc), :],
                send_sem=send_sems.at[_SEM_A0 + c],
                recv_sem=recv_sems.at[_SEM_A0 + c],
                device_id=ynbr,
                device_id_type=pl.DeviceIdType.MESH,
            )
            rdma.start()
            rdma_a.append(rdma)

        b_bf16[:, pl.ds(col1, nh)] = (
            b_ref[:, pl.ds(col1, nh)].astype(jnp.bfloat16)
        )
        acc[...] = jnp.dot(
            a_bf16[...], b_bf16[...], preferred_element_type=jnp.float32
        ).astype(jnp.bfloat16)

        rdma_b.wait_recv()
        rdma_fwd = pltpu.make_async_remote_copy(
            src_ref=b_nbr.at[:, pl.ds(col0, nh)],
            dst_ref=b_nbr.at[:, pl.ds(col0, nh)],
            send_sem=send_sems.at[_SEM_FWD],
            recv_sem=recv_sems.at[_SEM_FWD],
            device_id=xnbr,
            device_id_type=pl.DeviceIdType.MESH,
        )
        rdma_fwd.start()
        rdma_fwd.wait_recv()

        out_dmas = []
        for c in range(N_A_CHUNKS):
            o, mc = a_offs[c], A_CHUNK_ROWS[c]
            rdma_a[c].wait_recv()
            acc[pl.ds(o, mc), :] = (
                acc[pl.ds(o, mc), :]
                + jnp.dot(
                    a_nbr[pl.ds(o, mc), :], b_nbr[...],
                    preferred_element_type=jnp.float32,
                ).astype(jnp.bfloat16)
            )
            d = pltpu.make_async_copy(
                acc.at[pl.ds(o, mc), :], out_ref.at[pl.ds(o, mc), :],
                out_sems.at[c],
            )
            d.start()
            out_dmas.append(d)

        rdma_b.wait_send()
        for c in range(N_A_CHUNKS):
            rdma_a[c].wait_send()
        rdma_fwd.wait_send()
        for d in out_dmas:
            d.wait()

    return pl.pallas_call(
        body,
        out_shape=jax.ShapeDtypeStruct((m, n), jnp.bfloat16),
        in_specs=[
            pl.BlockSpec(memory_space=pltpu.VMEM),
            pl.BlockSpec(memory_space=pltpu.VMEM),
        ],
        out_specs=pl.BlockSpec(memory_space=pltpu.MemorySpace.HBM),
        scratch_shapes=[
            pltpu.VMEM((m, n), jnp.bfloat16),
            pltpu.VMEM((m, k), jnp.bfloat16),
            pltpu.VMEM((k, n), jnp.bfloat16),
            pltpu.VMEM((m, k), jnp.bfloat16),
            pltpu.VMEM((k, n), jnp.bfloat16),
            pltpu.SemaphoreType.DMA((_N_SEMS,)),
            pltpu.SemaphoreType.DMA((_N_SEMS,)),
            pltpu.SemaphoreType.DMA((N_A_CHUNKS,)),
        ],
        compiler_params=pltpu.CompilerParams(collective_id=0),
    )(A, B)


# device time: 50290 ns/iter; 1.0038x vs baseline; 1.0038x over previous
import jax
import jax.numpy as jnp
from jax import lax
from jax.experimental import pallas as pl
from jax.experimental.pallas import tpu as pltpu

A_CHUNK_ROWS = (512, 512, 384, 128)
N_A_CHUNKS = len(A_CHUNK_ROWS)

_SEM_B = 0
_SEM_A0 = 1
_SEM_FWD = _SEM_A0 + N_A_CHUNKS
_N_SEMS = _SEM_FWD + 1


def kernel(A, B):
    m, k = A.shape
    k2, n = B.shape
    assert k == k2
    assert sum(A_CHUNK_ROWS) == m
    a_offs = [sum(A_CHUNK_ROWS[:c]) for c in range(N_A_CHUNKS)]
    nh = n // 2

    def body(a_ref, b_ref, out_ref,
             a_bf16, b_bf16, a_nbr, b_nbr, send_sems, recv_sems):
        my_x = lax.axis_index("x")
        my_y = lax.axis_index("y")
        ynbr = (my_x, 1 - my_y)
        xnbr = (1 - my_x, my_y)
        col0 = my_x * nh
        col1 = (1 - my_x) * nh

        barrier_sem = pltpu.get_barrier_semaphore()
        for nbr in (ynbr, xnbr):
            pl.semaphore_signal(
                barrier_sem, inc=1,
                device_id=nbr, device_id_type=pl.DeviceIdType.MESH,
            )

        b_bf16[:, pl.ds(col0, nh)] = (
            b_ref[:, pl.ds(col0, nh)].astype(jnp.bfloat16)
        )
        pl.semaphore_wait(barrier_sem, 2)

        rdma_b = pltpu.make_async_remote_copy(
            src_ref=b_bf16.at[:, pl.ds(col0, nh)],
            dst_ref=b_nbr.at[:, pl.ds(col0, nh)],
            send_sem=send_sems.at[_SEM_B],
            recv_sem=recv_sems.at[_SEM_B],
            device_id=ynbr,
            device_id_type=pl.DeviceIdType.MESH,
        )
        rdma_b.start()

        rdma_a = []
        for c in range(N_A_CHUNKS):
            o, mc = a_offs[c], A_CHUNK_ROWS[c]
            a_bf16[pl.ds(o, mc), :] = (
                a_ref[pl.ds(o, mc), :].astype(jnp.bfloat16)
            )
            rdma = pltpu.make_async_remote_copy(
                src_ref=a_bf16.at[pl.ds(o, mc), :],
                dst_ref=a_nbr.at[pl.ds(o, mc), :],
                send_sem=send_sems.at[_SEM_A0 + c],
                recv_sem=recv_sems.at[_SEM_A0 + c],
                device_id=ynbr,
                device_id_type=pl.DeviceIdType.MESH,
            )
            rdma.start()
            rdma_a.append(rdma)

        b_bf16[:, pl.ds(col1, nh)] = (
            b_ref[:, pl.ds(col1, nh)].astype(jnp.bfloat16)
        )
        out_ref[...] = jnp.dot(
            a_bf16[...], b_bf16[...], preferred_element_type=jnp.float32
        ).astype(jnp.bfloat16)

        rdma_b.wait_recv()
        rdma_fwd = pltpu.make_async_remote_copy(
            src_ref=b_nbr.at[:, pl.ds(col0, nh)],
            dst_ref=b_nbr.at[:, pl.ds(col0, nh)],
            send_sem=send_sems.at[_SEM_FWD],
            recv_sem=recv_sems.at[_SEM_FWD],
            device_id=xnbr,
            device_id_type=pl.DeviceIdType.MESH,
        )
        rdma_fwd.start()
        rdma_fwd.wait_recv()

        for c in range(N_A_CHUNKS):
            o, mc = a_offs[c], A_CHUNK_ROWS[c]
            rdma_a[c].wait_recv()
            out_ref[pl.ds(o, mc), :] = (
                out_ref[pl.ds(o, mc), :]
                + jnp.dot(
                    a_nbr[pl.ds(o, mc), :], b_nbr[...],
                    preferred_element_type=jnp.float32,
                ).astype(jnp.bfloat16)
            )

        rdma_b.wait_send()
        for c in range(N_A_CHUNKS):
            rdma_a[c].wait_send()
        rdma_fwd.wait_send()

    return pl.pallas_call(
        body,
        out_shape=jax.ShapeDtypeStruct((m, n), jnp.bfloat16),
        in_specs=[
            pl.BlockSpec(memory_space=pltpu.VMEM),
            pl.BlockSpec(memory_space=pltpu.VMEM),
        ],
        out_specs=pl.BlockSpec(memory_space=pltpu.VMEM),
        scratch_shapes=[
            pltpu.VMEM((m, k), jnp.bfloat16),
            pltpu.VMEM((k, n), jnp.bfloat16),
            pltpu.VMEM((m, k), jnp.bfloat16),
            pltpu.VMEM((k, n), jnp.bfloat16),
            pltpu.SemaphoreType.DMA((_N_SEMS,)),
            pltpu.SemaphoreType.DMA((_N_SEMS,)),
        ],
        compiler_params=pltpu.CompilerParams(collective_id=0),
    )(A, B)
